# baseline (speedup 1.0000x reference)
import functools

import numpy as np
import jax
import jax.numpy as jnp
from jax.experimental import pallas as pl
from jax.experimental.pallas import tpu as pltpu
from jax.experimental.pallas import tpu_sc as plsc

_B = 4096
_E = 128
_H = 16
_HD = 8
_NE = 64
_TOPK = 8
_ALPHA = 0.7
_BLK = 4096
_GRID = _B // _BLK
_INV_SQRT_HD = float(1.0 / np.sqrt(_HD))
_NEG_INF = float("-inf")

_NC = 2
_NS = 16
_NW = _NC * _NS
_RPW = _B // _NW


def _moe_logits_body(tid_ref, nl_ref, tbl_ref, nwrow_ref, nb_ref, wqt_ref, bq_ref,
                     wk_ref, ekt_ref, bkcol_ref, gwt_ref, gb_ref,
                     logits_ref, load_ref, kt_ref):
    i = pl.program_id(0)

    @pl.when(i == 0)
    def _prep():
        kt_ref[...] = jnp.dot(wk_ref[...], ekt_ref[...],
                              preferred_element_type=jnp.float32) + bkcol_ref[...]
        load_ref[...] = jnp.zeros_like(load_ref)

    tid = tid_ref[...]
    onehot = (tid == jax.lax.broadcasted_iota(jnp.int32, (_BLK, 8), 1)
              ).astype(jnp.float32)
    task_embed = jnp.dot(onehot, tbl_ref[...],
                         preferred_element_type=jnp.float32)
    noise_embed = nl_ref[...] * nwrow_ref[...] + nb_ref[...]
    query = _ALPHA * task_embed + (1.0 - _ALPHA) * noise_embed
    q = jnp.dot(query, wqt_ref[...],
                preferred_element_type=jnp.float32) + bq_ref[...]
    kT = kt_ref[...]

    ones_e = jnp.ones((_E, 1), jnp.float32)
    acc = jnp.zeros((_BLK, _E), jnp.float32)
    for h in range(_H):
        s = jnp.dot(q[:, h * _HD:(h + 1) * _HD], kT[h * _HD:(h + 1) * _HD, :],
                    preferred_element_type=jnp.float32) * _INV_SQRT_HD
        p = jnp.exp(s)
        ssum = jnp.dot(p, ones_e, preferred_element_type=jnp.float32)
        acc = acc + p * (1.0 / ssum)
    ew = jnp.exp(acc * (1.0 / _H))
    esum = jnp.dot(ew, ones_e, preferred_element_type=jnp.float32)
    ew = ew * (1.0 / esum)
    logits = jnp.dot(ew, gwt_ref[...],
                     preferred_element_type=jnp.float32) + gb_ref[...]
    logits_ref[...] = logits

    ones_n = jnp.ones((_NE, 1), jnp.float32)
    work = logits
    cum = jnp.zeros((_BLK, 1), jnp.float32)
    thr = jnp.zeros((_BLK, 1), jnp.float32)
    need = jnp.zeros((_BLK, 1), jnp.float32)
    for t in range(_TOPK):
        m = jnp.max(work, axis=-1, keepdims=True)
        eq = work == m
        c = jnp.dot(eq.astype(jnp.float32), ones_n,
                    preferred_element_type=jnp.float32)
        active = cum < float(_TOPK)
        found = jnp.logical_and(active, cum + c >= float(_TOPK))
        thr = jnp.where(found, m, thr)
        need = jnp.where(found, float(_TOPK) - cum, need)
        cum = cum + jnp.where(active, c, 0.0)
        work = jnp.where(eq, _NEG_INF, work)
    eqthr = logits == thr
    lt_tri = (jax.lax.broadcasted_iota(jnp.int32, (_NE, _NE), 0)
              < jax.lax.broadcasted_iota(jnp.int32, (_NE, _NE), 1)
              ).astype(jnp.float32)
    pceq = jnp.dot(eqthr.astype(jnp.float32), lt_tri,
                   preferred_element_type=jnp.float32)
    sel = jnp.logical_or(logits > thr,
                         jnp.logical_and(eqthr, pceq < need))
    pe = jnp.where(sel, jnp.exp(logits - thr), 0.0)
    z = jnp.dot(pe, ones_n, preferred_element_type=jnp.float32)
    gates = pe * (1.0 / z)
    load_ref[...] += jnp.sum(gates, axis=0, keepdims=True)


def _moe_logits_call(tid, nl, tbl, nwrow, nb, wqt, bq, wk, ekt, bkcol, gwt, gb):
    nrows = tid.shape[0]
    row = lambda i: (i, 0)
    fixed = lambda i: (0, 0)
    return pl.pallas_call(
        _moe_logits_body,
        grid=(nrows // _BLK,),
        in_specs=[
            pl.BlockSpec((_BLK, 1), row),
            pl.BlockSpec((_BLK, 1), row),
            pl.BlockSpec((8, _E), fixed),
            pl.BlockSpec((1, _E), fixed),
            pl.BlockSpec((1, _E), fixed),
            pl.BlockSpec((_E, _E), fixed),
            pl.BlockSpec((1, _E), fixed),
            pl.BlockSpec((_E, _E), fixed),
            pl.BlockSpec((_E, _E), fixed),
            pl.BlockSpec((_E, 1), fixed),
            pl.BlockSpec((_E, _NE), fixed),
            pl.BlockSpec((1, _NE), fixed),
        ],
        out_specs=[
            pl.BlockSpec((_BLK, _NE), row),
            pl.BlockSpec((1, _NE), fixed),
        ],
        out_shape=[
            jax.ShapeDtypeStruct((nrows, _NE), jnp.float32),
            jax.ShapeDtypeStruct((1, _NE), jnp.float32),
        ],
        scratch_shapes=[pltpu.VMEM((_E, _E), jnp.float32)],
        compiler_params=pltpu.CompilerParams(
            dimension_semantics=("arbitrary",),
            vmem_limit_bytes=100 * 1024 * 1024,
        ),
    )(tid, nl, tbl, nwrow, nb, wqt, bq, wk, ekt, bkcol, gwt, gb)


def _sc_gates_body(rpw, logits_hbm, gates_hbm, lg_ref, gt_ref):
    c = jax.lax.axis_index("c")
    s = jax.lax.axis_index("s")
    wid = s * _NC + c
    base = wid * rpw
    pltpu.sync_copy(logits_hbm.at[pl.ds(base, rpw)], lg_ref)

    lane = jax.lax.iota(jnp.int32, 16)
    topmask = lane < _TOPK
    zeros16 = jnp.zeros((16,), jnp.float32)

    def _zero_row(r, carry):
        gt_ref[r, pl.ds(0, 16)] = zeros16
        gt_ref[r, pl.ds(16, 16)] = zeros16
        gt_ref[r, pl.ds(32, 16)] = zeros16
        gt_ref[r, pl.ds(48, 16)] = zeros16
        return carry

    jax.lax.fori_loop(0, rpw, _zero_row, 0)

    def _merge_top(ka, va, kb, vb):
        rk = jax.lax.rev(kb, (0,))
        rv = jax.lax.rev(vb, (0,))
        take_a = jnp.logical_or(ka > rk, jnp.logical_and(ka == rk, va < rv))
        mk = jnp.where(take_a, ka, rk)
        mv = jnp.where(take_a, va, rv)
        return plsc.sort_key_val(mk, mv, descending=True)

    @plsc.parallel_loop(0, rpw, 1, unroll=8)
    def _row(r):
        k0 = lg_ref[r, pl.ds(0, 16)]
        k1 = lg_ref[r, pl.ds(16, 16)]
        k2 = lg_ref[r, pl.ds(32, 16)]
        k3 = lg_ref[r, pl.ds(48, 16)]
        a_k, a_v = plsc.sort_key_val(k0, lane, descending=True)
        b_k, b_v = plsc.sort_key_val(k1, lane + 16, descending=True)
        c_k, c_v = plsc.sort_key_val(k2, lane + 32, descending=True)
        d_k, d_v = plsc.sort_key_val(k3, lane + 48, descending=True)
        m1k, m1v = _merge_top(a_k, a_v, b_k, b_v)
        m2k, m2v = _merge_top(c_k, c_v, d_k, d_v)
        tk, tv = _merge_top(m1k, m1v, m2k, m2v)
        kmax = jnp.max(tk)
        e = jnp.where(topmask, jnp.exp(tk - kmax), 0.0)
        zsum = jnp.sum(e)
        g = e / jnp.broadcast_to(zsum, (16,))
        ridx = jnp.broadcast_to(r, (16,)).astype(jnp.int32)
        plsc.store_scatter(gt_ref, [ridx, tv], g, mask=topmask)

    pltpu.sync_copy(gt_ref, gates_hbm.at[pl.ds(base, rpw)])


def _sc_gates_call(logits):
    nrows = logits.shape[0]
    rpw = nrows // _NW
    mesh = plsc.VectorSubcoreMesh(core_axis_name="c", subcore_axis_name="s")
    return pl.kernel(
        functools.partial(_sc_gates_body, rpw),
        out_type=jax.ShapeDtypeStruct((nrows, _NE), jnp.float32),
        mesh=mesh,
        scratch_types=[
            pltpu.VMEM((rpw, _NE), jnp.float32),
            pltpu.VMEM((rpw, _NE), jnp.float32),
        ],
        compiler_params=pltpu.CompilerParams(needs_layout_passes=False),
    )(logits)


@jax.jit
def _impl(taskID, noise_level, task_embed_table, noise_W, noise_b, expert_keys,
          in_proj_W, in_proj_b, gate_W, gate_b):
    tid = taskID.astype(jnp.int32).reshape(_B, 1)
    nl = noise_level.reshape(_B, 1)
    tbl = jnp.zeros((8, _E), jnp.float32).at[:5, :].set(task_embed_table)
    nwrow = noise_W.reshape(1, _E)
    nb = noise_b.reshape(1, _E)
    wqt = in_proj_W[:_E].T
    bq = in_proj_b[:_E].reshape(1, _E)
    wk = in_proj_W[_E:2 * _E]
    ekt = expert_keys.T
    bkcol = in_proj_b[_E:2 * _E].reshape(_E, 1)
    gwt = gate_W.T
    gb = gate_b.reshape(1, _NE)
    logits, load = _moe_logits_call(tid, nl, tbl, nwrow, nb, wqt, bq, wk, ekt,
                                    bkcol, gwt, gb)
    gates = _sc_gates_call(logits)
    return gates, load.reshape(_NE)


def kernel(taskID, noise_level, task_embed_table, noise_W, noise_b, expert_keys,
           in_proj_W, in_proj_b, out_proj_W, out_proj_b, gate_W, gate_b, train):
    del out_proj_W, out_proj_b, train
    return _impl(taskID, noise_level, task_embed_table, noise_W, noise_b,
                 expert_keys, in_proj_W, in_proj_b, gate_W, gate_b)


# speedup vs baseline: 1.0127x; 1.0127x over previous
import functools

import numpy as np
import jax
import jax.numpy as jnp
from jax.experimental import pallas as pl
from jax.experimental.pallas import tpu as pltpu
from jax.experimental.pallas import tpu_sc as plsc

_B = 4096
_E = 128
_H = 16
_HD = 8
_NE = 64
_TOPK = 8
_ALPHA = 0.7
_BLK = 4096
_GRID = _B // _BLK
_INV_SQRT_HD = float(1.0 / np.sqrt(_HD))
_NEG_INF = float("-inf")

_NC = 2
_NS = 16
_NW = _NC * _NS
_RPW = _B // _NW


def _moe_logits_body(tid_ref, nl_ref, tbl_ref, nwrow_ref, nb_ref, wqt_ref, bq_ref,
                     wk_ref, ekt_ref, bkcol_ref, gwt_ref, gb_ref,
                     logits_ref, load_ref, kt_ref):
    i = pl.program_id(0)

    @pl.when(i == 0)
    def _prep():
        kt_ref[...] = jnp.dot(wk_ref[...], ekt_ref[...],
                              preferred_element_type=jnp.float32) + bkcol_ref[...]
        load_ref[...] = jnp.zeros_like(load_ref)

    tid = tid_ref[...]
    onehot = (tid == jax.lax.broadcasted_iota(jnp.int32, (_BLK, 8), 1)
              ).astype(jnp.float32)
    task_embed = jnp.dot(onehot, tbl_ref[...],
                         preferred_element_type=jnp.float32)
    noise_embed = nl_ref[...] * nwrow_ref[...] + nb_ref[...]
    query = _ALPHA * task_embed + (1.0 - _ALPHA) * noise_embed
    q = jnp.dot(query, wqt_ref[...],
                preferred_element_type=jnp.float32) + bq_ref[...]
    kT = kt_ref[...]

    ones_e = jnp.ones((_E, 1), jnp.float32)
    acc = jnp.zeros((_BLK, _E), jnp.float32)
    for h in range(_H):
        s = jnp.dot(q[:, h * _HD:(h + 1) * _HD], kT[h * _HD:(h + 1) * _HD, :],
                    preferred_element_type=jnp.float32) * _INV_SQRT_HD
        p = jnp.exp(s)
        ssum = jnp.dot(p, ones_e, preferred_element_type=jnp.float32)
        acc = acc + p * (1.0 / ssum)
    ew = jnp.exp(acc * (1.0 / _H))
    esum = jnp.dot(ew, ones_e, preferred_element_type=jnp.float32)
    ew = ew * (1.0 / esum)
    logits = jnp.dot(ew, gwt_ref[...],
                     preferred_element_type=jnp.float32) + gb_ref[...]
    logits_ref[...] = logits

    ones_n = jnp.ones((_NE, 1), jnp.float32)
    work = logits
    cum = jnp.zeros((_BLK, 1), jnp.float32)
    thr = jnp.zeros((_BLK, 1), jnp.float32)
    need = jnp.zeros((_BLK, 1), jnp.float32)
    for t in range(_TOPK):
        m = jnp.max(work, axis=-1, keepdims=True)
        eq = work == m
        c = jnp.dot(eq.astype(jnp.float32), ones_n,
                    preferred_element_type=jnp.float32)
        active = cum < float(_TOPK)
        found = jnp.logical_and(active, cum + c >= float(_TOPK))
        thr = jnp.where(found, m, thr)
        need = jnp.where(found, float(_TOPK) - cum, need)
        cum = cum + jnp.where(active, c, 0.0)
        work = jnp.where(eq, _NEG_INF, work)
    eqthr = logits == thr
    lt_tri = (jax.lax.broadcasted_iota(jnp.int32, (_NE, _NE), 0)
              < jax.lax.broadcasted_iota(jnp.int32, (_NE, _NE), 1)
              ).astype(jnp.float32)
    pceq = jnp.dot(eqthr.astype(jnp.float32), lt_tri,
                   preferred_element_type=jnp.float32)
    sel = jnp.logical_or(logits > thr,
                         jnp.logical_and(eqthr, pceq < need))
    pe = jnp.where(sel, jnp.exp(logits - thr), 0.0)
    z = jnp.dot(pe, ones_n, preferred_element_type=jnp.float32)
    gates = pe * (1.0 / z)
    load_ref[...] += jnp.sum(gates, axis=0, keepdims=True)


def _moe_logits_call(tid, nl, tbl, nwrow, nb, wqt, bq, wk, ekt, bkcol, gwt, gb):
    nrows = tid.shape[0]
    row = lambda i: (i, 0)
    fixed = lambda i: (0, 0)
    return pl.pallas_call(
        _moe_logits_body,
        grid=(nrows // _BLK,),
        in_specs=[
            pl.BlockSpec((_BLK, 1), row),
            pl.BlockSpec((_BLK, 1), row),
            pl.BlockSpec((8, _E), fixed),
            pl.BlockSpec((1, _E), fixed),
            pl.BlockSpec((1, _E), fixed),
            pl.BlockSpec((_E, _E), fixed),
            pl.BlockSpec((1, _E), fixed),
            pl.BlockSpec((_E, _E), fixed),
            pl.BlockSpec((_E, _E), fixed),
            pl.BlockSpec((_E, 1), fixed),
            pl.BlockSpec((_E, _NE), fixed),
            pl.BlockSpec((1, _NE), fixed),
        ],
        out_specs=[
            pl.BlockSpec((_BLK, _NE), row),
            pl.BlockSpec((1, _NE), fixed),
        ],
        out_shape=[
            jax.ShapeDtypeStruct((nrows, _NE), jnp.float32),
            jax.ShapeDtypeStruct((1, _NE), jnp.float32),
        ],
        scratch_shapes=[pltpu.VMEM((_E, _E), jnp.float32)],
        compiler_params=pltpu.CompilerParams(
            dimension_semantics=("arbitrary",),
            vmem_limit_bytes=100 * 1024 * 1024,
        ),
    )(tid, nl, tbl, nwrow, nb, wqt, bq, wk, ekt, bkcol, gwt, gb)


def _sc_gates_body(rpw, logits_hbm, gates_hbm, lg_ref, gt_ref):
    c = jax.lax.axis_index("c")
    s = jax.lax.axis_index("s")
    wid = s * _NC + c
    base = wid * rpw
    pltpu.sync_copy(logits_hbm.at[pl.ds(base, rpw)], lg_ref)

    lane = jax.lax.iota(jnp.int32, 16)
    topmask = lane < _TOPK
    zeros16 = jnp.zeros((16,), jnp.float32)

    def _zero_row(r, carry):
        gt_ref[r, pl.ds(0, 16)] = zeros16
        gt_ref[r, pl.ds(16, 16)] = zeros16
        gt_ref[r, pl.ds(32, 16)] = zeros16
        gt_ref[r, pl.ds(48, 16)] = zeros16
        return carry

    jax.lax.fori_loop(0, rpw, _zero_row, 0)

    def _merge_top(ka, va, kb, vb):
        rk = jax.lax.rev(kb, (0,))
        rv = jax.lax.rev(vb, (0,))
        take_a = jnp.logical_or(ka > rk, jnp.logical_and(ka == rk, va < rv))
        mk = jnp.where(take_a, ka, rk)
        mv = jnp.where(take_a, va, rv)
        return plsc.sort_key_val(mk, mv, descending=True)

    @plsc.parallel_loop(0, rpw, 1, unroll=4)
    def _row(r):
        k0 = lg_ref[r, pl.ds(0, 16)]
        k1 = lg_ref[r, pl.ds(16, 16)]
        k2 = lg_ref[r, pl.ds(32, 16)]
        k3 = lg_ref[r, pl.ds(48, 16)]
        a_k, a_v = plsc.sort_key_val(k0, lane, descending=True)
        b_k, b_v = plsc.sort_key_val(k1, lane + 16, descending=True)
        c_k, c_v = plsc.sort_key_val(k2, lane + 32, descending=True)
        d_k, d_v = plsc.sort_key_val(k3, lane + 48, descending=True)
        m1k, m1v = _merge_top(a_k, a_v, b_k, b_v)
        m2k, m2v = _merge_top(c_k, c_v, d_k, d_v)
        tk, tv = _merge_top(m1k, m1v, m2k, m2v)
        kmax = jnp.max(tk)
        e = jnp.where(topmask, jnp.exp(tk - kmax), 0.0)
        zsum = jnp.sum(e)
        g = e / jnp.broadcast_to(zsum, (16,))
        ridx = jnp.broadcast_to(r, (16,)).astype(jnp.int32)
        plsc.store_scatter(gt_ref, [ridx, tv], g, mask=topmask)

    pltpu.sync_copy(gt_ref, gates_hbm.at[pl.ds(base, rpw)])


def _sc_gates_call(logits):
    nrows = logits.shape[0]
    rpw = nrows // _NW
    mesh = plsc.VectorSubcoreMesh(core_axis_name="c", subcore_axis_name="s")
    return pl.kernel(
        functools.partial(_sc_gates_body, rpw),
        out_type=jax.ShapeDtypeStruct((nrows, _NE), jnp.float32),
        mesh=mesh,
        scratch_types=[
            pltpu.VMEM((rpw, _NE), jnp.float32),
            pltpu.VMEM((rpw, _NE), jnp.float32),
        ],
        compiler_params=pltpu.CompilerParams(needs_layout_passes=False),
    )(logits)


@jax.jit
def _impl(taskID, noise_level, task_embed_table, noise_W, noise_b, expert_keys,
          in_proj_W, in_proj_b, gate_W, gate_b):
    tid = taskID.astype(jnp.int32).reshape(_B, 1)
    nl = noise_level.reshape(_B, 1)
    tbl = jnp.zeros((8, _E), jnp.float32).at[:5, :].set(task_embed_table)
    nwrow = noise_W.reshape(1, _E)
    nb = noise_b.reshape(1, _E)
    wqt = in_proj_W[:_E].T
    bq = in_proj_b[:_E].reshape(1, _E)
    wk = in_proj_W[_E:2 * _E]
    ekt = expert_keys.T
    bkcol = in_proj_b[_E:2 * _E].reshape(_E, 1)
    gwt = gate_W.T
    gb = gate_b.reshape(1, _NE)
    logits, load = _moe_logits_call(tid, nl, tbl, nwrow, nb, wqt, bq, wk, ekt,
                                    bkcol, gwt, gb)
    gates = _sc_gates_call(logits)
    return gates, load.reshape(_NE)


def kernel(taskID, noise_level, task_embed_table, noise_W, noise_b, expert_keys,
           in_proj_W, in_proj_b, out_proj_W, out_proj_b, gate_W, gate_b, train):
    del out_proj_W, out_proj_b, train
    return _impl(taskID, noise_level, task_embed_table, noise_W, noise_b,
                 expert_keys, in_proj_W, in_proj_b, gate_W, gate_b)
